# baseline (speedup 1.0000x reference)
import dataclasses
import functools

import jax
import jax.numpy as jnp
from jax import lax
from jax.experimental import pallas as pl
from jax.experimental.pallas import tpu as pltpu
from jax.experimental.pallas import tpu_sc as plsc

_NC = 2
_NS = 16
_NW = _NC * _NS
_LANES = 16


def _vector_mesh():
    return plsc.VectorSubcoreMesh(core_axis_name="c", subcore_axis_name="s")


def _sc_compiler_params():
    cp = pltpu.CompilerParams()
    if "needs_layout_passes" in pltpu.CompilerParams.__dataclass_fields__:
        cp = dataclasses.replace(cp, needs_layout_passes=False)
    return cp


def _deg_partials(dst_r, n_nodes):
    _, nchunk, chunk = dst_r.shape

    @functools.partial(
        pl.kernel,
        mesh=_vector_mesh(),
        out_type=jax.ShapeDtypeStruct((_NW, 1, n_nodes), jnp.float32),
        compiler_params=_sc_compiler_params(),
        scratch_types=[
            pltpu.VMEM((nchunk, chunk), jnp.int32),
            pltpu.VMEM((1, n_nodes), jnp.float32),
        ],
    )
    def k(dst_hbm, out_hbm, idx_v, hist_v):
        cid = lax.axis_index("c")
        sid = lax.axis_index("s")
        wid = sid * _NC + cid
        pltpu.sync_copy(dst_hbm.at[wid], idx_v)

        @pl.loop(0, n_nodes, step=_LANES)
        def _(i):
            hist_v[0, pl.ds(i, _LANES)] = jnp.zeros((_LANES,), jnp.float32)

        ones = jnp.ones((_LANES,), jnp.float32)
        zrow = jnp.zeros((_LANES,), jnp.int32)

        @pl.loop(0, nchunk)
        def _(j):
            @pl.loop(0, chunk, step=_LANES)
            def _(kk):
                idx = idx_v[j, pl.ds(kk, _LANES)]
                plsc.addupdate_scatter(hist_v, [zrow, idx], ones)

        pltpu.sync_copy(hist_v, out_hbm.at[wid])

    return k(dst_r)


def _sc_aggregate(y, src_r, dst_r, n_nodes):
    _, nblk, bchunk, chunk = dst_r.shape
    d = y.shape[1]
    zb = 80
    rpt_a = -(-(n_nodes // _NS) // zb) * zb
    last_rows = n_nodes - (_NS - 1) * rpt_a

    @functools.partial(
        pl.kernel,
        mesh=_vector_mesh(),
        out_type=jax.ShapeDtypeStruct((_NC, n_nodes, d), jnp.float32),
        scratch_types=[
            pltpu.VMEM((bchunk, chunk), jnp.int32),
            pltpu.VMEM((bchunk, chunk), jnp.int32),
            pltpu.VMEM((bchunk, chunk), jnp.int32),
            pltpu.VMEM((bchunk, chunk), jnp.int32),
            pltpu.VMEM((chunk, d), jnp.float32),
            pltpu.VMEM((chunk, d), jnp.float32),
            pltpu.VMEM_SHARED((n_nodes, d), jnp.float32),
            pltpu.SemaphoreType.DMA,
            pltpu.SemaphoreType.DMA,
        ],
    )
    def k(y_hbm, src_hbm, dst_hbm, out_hbm, src_ia, dst_ia, src_ib, dst_ib,
          rows_a, rows_b, acc_sh, gsem, isem):
        cid = lax.axis_index("c")
        sid = lax.axis_index("s")
        wid = sid * _NC + cid

        row0 = sid * rpt_a
        my_rows = jnp.where(sid == _NS - 1, last_rows, rpt_a)

        @pl.loop(0, zb)
        def _(r):
            @pl.loop(0, d, step=_LANES)
            def _(cc):
                rows_a[r, pl.ds(cc, _LANES)] = jnp.zeros((_LANES,), jnp.float32)

        @pl.loop(0, my_rows, step=zb)
        def _(r):
            pltpu.sync_copy(rows_a.at[pl.ds(0, zb)],
                            acc_sh.at[pl.ds(row0 + r, zb)])

        plsc.subcore_barrier()

        def load_idx(b, src_i, dst_i):
            pltpu.async_copy(src_hbm.at[wid, b], src_i, isem)
            pltpu.async_copy(dst_hbm.at[wid, b], dst_i, isem)

        def wait_idx(src_i, dst_i):
            pltpu.make_async_copy(src_hbm.at[wid, 0], src_i, isem).wait()
            pltpu.make_async_copy(dst_hbm.at[wid, 0], dst_i, isem).wait()

        def process(src_i, dst_i):
            def start_gather(j, buf):
                pltpu.async_copy(y_hbm.at[src_i.at[j]], buf, gsem)

            def wait_gather(buf):
                pltpu.make_async_copy(y_hbm.at[src_i.at[0]], buf, gsem).wait()

            def scatter(j, buf):
                pltpu.sync_copy(buf, acc_sh.at[dst_i.at[j]], add=True)

            start_gather(0, rows_a)

            @pl.loop(0, (bchunk - 2) // 2)
            def _(j2):
                c0 = 2 * j2
                wait_gather(rows_a)
                start_gather(c0 + 1, rows_b)
                scatter(c0, rows_a)
                wait_gather(rows_b)
                start_gather(c0 + 2, rows_a)
                scatter(c0 + 1, rows_b)

            wait_gather(rows_a)
            start_gather(bchunk - 1, rows_b)
            scatter(bchunk - 2, rows_a)
            wait_gather(rows_b)
            scatter(bchunk - 1, rows_b)

        pltpu.sync_copy(src_hbm.at[wid, 0], src_ia)
        pltpu.sync_copy(dst_hbm.at[wid, 0], dst_ia)
        load_idx(1, src_ib, dst_ib)

        @pl.loop(0, nblk // 2)
        def _(p):
            b0 = 2 * p
            process(src_ia, dst_ia)
            wait_idx(src_ib, dst_ib)

            @pl.when(b0 + 2 < nblk)
            def _():
                load_idx(b0 + 2, src_ia, dst_ia)

            process(src_ib, dst_ib)

            @pl.when(b0 + 2 < nblk)
            def _():
                wait_idx(src_ia, dst_ia)

            @pl.when(b0 + 3 < nblk)
            def _():
                load_idx(b0 + 3, src_ib, dst_ib)

        plsc.subcore_barrier()

        @pl.loop(0, my_rows, step=zb)
        def _(r):
            pltpu.sync_copy(acc_sh.at[pl.ds(row0 + r, zb)],
                            rows_a.at[pl.ds(0, zb)])
            pltpu.sync_copy(rows_a.at[pl.ds(0, zb)],
                            out_hbm.at[cid, pl.ds(row0 + r, zb)])

    return k(y, src_r, dst_r)


def _tc_inv_prescale(degp, t):
    n, d = t.shape

    def body(degp_ref, t_ref, inv_ref, y_ref):
        ones = jnp.ones((_NW, 1), jnp.float32)
        deg = lax.dot_general(degp_ref[...], ones, (((0,), (0,)), ((), ())),
                              precision=lax.Precision.HIGHEST,
                              preferred_element_type=jnp.float32)
        inv = lax.rsqrt(jnp.maximum(deg, 1.0))
        inv_ref[...] = inv
        y_ref[pl.ds(0, n), :] = t_ref[...] * inv
        y_ref[pl.ds(n, 8), :] = jnp.zeros((8, d), jnp.float32)

    return pl.pallas_call(
        body,
        out_shape=[jax.ShapeDtypeStruct((n, 1), jnp.float32),
                   jax.ShapeDtypeStruct((n + 8, d), jnp.float32)],
    )(degp, t)


def _tc_layer_mid(p, inv, t, w, b):
    n, d = t.shape

    def body(p_ref, inv_ref, t_ref, w_ref, b_ref, h_ref, y_ref):
        inv = inv_ref[...]
        agg = (p_ref[0] + p_ref[1]) * inv
        z = lax.dot_general(agg, w_ref[...], (((1,), (0,)), ((), ())),
                            precision=lax.Precision.HIGHEST,
                            preferred_element_type=jnp.float32)
        h = jnp.maximum(z + b_ref[...] + t_ref[...], 0.0)
        h_ref[...] = h
        y_ref[pl.ds(0, n), :] = h * inv
        y_ref[pl.ds(n, 8), :] = jnp.zeros((8, d), jnp.float32)

    return pl.pallas_call(
        body,
        out_shape=[jax.ShapeDtypeStruct((n, d), jnp.float32),
                   jax.ShapeDtypeStruct((n + 8, d), jnp.float32)],
    )(p, inv, t, w, b.reshape(1, d))


def _tc_layer_out(p, inv, h_prev, w, b):
    n, d = h_prev.shape

    def body(p_ref, inv_ref, h_ref, w_ref, b_ref, o_ref):
        agg = (p_ref[0] + p_ref[1]) * inv_ref[...]
        z = lax.dot_general(agg, w_ref[...], (((1,), (0,)), ((), ())),
                            precision=lax.Precision.HIGHEST,
                            preferred_element_type=jnp.float32)
        o_ref[...] = z + b_ref[...] + h_ref[...]

    return pl.pallas_call(
        body,
        out_shape=jax.ShapeDtypeStruct((n, d), jnp.float32),
    )(p, inv, h_prev, w, b.reshape(1, d))


def kernel(T, edge_index, W1, b1, W2, b2):
    n, d = T.shape
    e = edge_index.shape[1]
    chunk = 128
    bchunk = 20
    epw = e // _NW
    blk_edges = bchunk * chunk
    epw_pad = -(-epw // (2 * blk_edges)) * (2 * blk_edges)
    nblk = epw_pad // blk_edges
    pad_n = epw_pad - epw
    w_ids = jnp.arange(_NW, dtype=jnp.int32)
    pad_i = jnp.arange(pad_n, dtype=jnp.int32)
    src_pad = jnp.broadcast_to(n + (pad_i % 8)[None, :], (_NW, pad_n))
    dst_pad = (w_ids[:, None] * 997 + pad_i[None, :] * 61) % n
    src_w = jnp.concatenate(
        [edge_index[0].reshape(_NW, epw), src_pad.astype(jnp.int32)], axis=1)
    dst_w = jnp.concatenate(
        [edge_index[1].reshape(_NW, epw), dst_pad.astype(jnp.int32)], axis=1)
    src_r = src_w.reshape(_NW, nblk, bchunk, chunk)
    dst_r = dst_w.reshape(_NW, nblk, bchunk, chunk)
    dst_deg = edge_index[1].reshape(_NW, e // (_NW * _LANES), _LANES)

    degp = _deg_partials(dst_deg, n).reshape(_NW, n)
    inv, y1 = _tc_inv_prescale(degp, T)
    p1 = _sc_aggregate(y1, src_r, dst_r, n)
    h1, y2 = _tc_layer_mid(p1, inv, T, W1, b1)
    p2 = _sc_aggregate(y2, src_r, dst_r, n)
    return _tc_layer_out(p2, inv, h1, W2, b2)


# speedup vs baseline: 1.1360x; 1.1360x over previous
import dataclasses
import functools

import jax
import jax.numpy as jnp
from jax import lax
from jax.experimental import pallas as pl
from jax.experimental.pallas import tpu as pltpu
from jax.experimental.pallas import tpu_sc as plsc

_NC = 2
_NS = 16
_NW = _NC * _NS
_LANES = 16


def _vector_mesh():
    return plsc.VectorSubcoreMesh(core_axis_name="c", subcore_axis_name="s")


def _sc_compiler_params():
    cp = pltpu.CompilerParams()
    if "needs_layout_passes" in pltpu.CompilerParams.__dataclass_fields__:
        cp = dataclasses.replace(cp, needs_layout_passes=False)
    return cp


def _deg_partials(dst_r, n_nodes):
    _, nchunk, chunk = dst_r.shape

    @functools.partial(
        pl.kernel,
        mesh=_vector_mesh(),
        out_type=jax.ShapeDtypeStruct((_NW, 1, n_nodes), jnp.float32),
        compiler_params=_sc_compiler_params(),
        scratch_types=[
            pltpu.VMEM((nchunk, chunk), jnp.int32),
            pltpu.VMEM((1, n_nodes), jnp.float32),
        ],
    )
    def k(dst_hbm, out_hbm, idx_v, hist_v):
        cid = lax.axis_index("c")
        sid = lax.axis_index("s")
        wid = sid * _NC + cid
        pltpu.sync_copy(dst_hbm.at[wid], idx_v)

        @pl.loop(0, n_nodes, step=_LANES)
        def _(i):
            hist_v[0, pl.ds(i, _LANES)] = jnp.zeros((_LANES,), jnp.float32)

        ones = jnp.ones((_LANES,), jnp.float32)
        zrow = jnp.zeros((_LANES,), jnp.int32)

        @pl.loop(0, nchunk)
        def _(j):
            @pl.loop(0, chunk, step=_LANES)
            def _(kk):
                idx = idx_v[j, pl.ds(kk, _LANES)]
                plsc.addupdate_scatter(hist_v, [zrow, idx], ones)

        pltpu.sync_copy(hist_v, out_hbm.at[wid])

    return k(dst_r)


def _sc_aggregate(y, src_r, dst_r, n_nodes):
    _, nblk, bchunk, chunk = dst_r.shape
    d = y.shape[1]
    zb = 80
    rpt_a = -(-(n_nodes // _NS) // zb) * zb
    last_rows = n_nodes - (_NS - 1) * rpt_a
    n_acc = n_nodes + _NS

    @functools.partial(
        pl.kernel,
        mesh=_vector_mesh(),
        out_type=jax.ShapeDtypeStruct((_NC, n_nodes, d), jnp.float32),
        scratch_types=[
            pltpu.VMEM((bchunk, chunk), jnp.int32),
            pltpu.VMEM((bchunk, chunk), jnp.int32),
            pltpu.VMEM((chunk, d), jnp.float32),
            pltpu.VMEM((chunk, d), jnp.float32),
            pltpu.VMEM_SHARED((n_acc, d), jnp.float32),
            pltpu.SemaphoreType.DMA,
        ],
    )
    def k(y_hbm, src_hbm, dst_hbm, out_hbm, src_i, dst_i,
          rows_a, rows_b, acc_sh, gsem):
        cid = lax.axis_index("c")
        sid = lax.axis_index("s")
        wid = sid * _NC + cid

        row0 = sid * rpt_a
        my_rows = jnp.where(sid == _NS - 1, last_rows, rpt_a)

        @pl.loop(0, zb)
        def _(r):
            @pl.loop(0, d, step=_LANES)
            def _(cc):
                rows_a[r, pl.ds(cc, _LANES)] = jnp.zeros((_LANES,), jnp.float32)

        @pl.loop(0, my_rows, step=zb)
        def _(r):
            pltpu.sync_copy(rows_a.at[pl.ds(0, zb)],
                            acc_sh.at[pl.ds(row0 + r, zb)])

        plsc.subcore_barrier()

        def start_gather(j, buf):
            pltpu.async_copy(y_hbm.at[src_i.at[j]], buf, gsem)

        def wait_gather(buf):
            pltpu.make_async_copy(y_hbm.at[src_i.at[0]], buf, gsem).wait()

        def scatter(j, buf):
            pltpu.sync_copy(buf, acc_sh.at[dst_i.at[j]], add=True)

        @pl.loop(0, nblk)
        def _(b):
            pltpu.sync_copy(src_hbm.at[wid, b], src_i)
            pltpu.sync_copy(dst_hbm.at[wid, b], dst_i)
            start_gather(0, rows_a)

            @pl.loop(0, (bchunk - 2) // 2)
            def _(j2):
                c0 = 2 * j2
                wait_gather(rows_a)
                start_gather(c0 + 1, rows_b)
                scatter(c0, rows_a)
                wait_gather(rows_b)
                start_gather(c0 + 2, rows_a)
                scatter(c0 + 1, rows_b)

            wait_gather(rows_a)
            start_gather(bchunk - 1, rows_b)
            scatter(bchunk - 2, rows_a)
            wait_gather(rows_b)
            scatter(bchunk - 1, rows_b)

        plsc.subcore_barrier()

        @pl.loop(0, my_rows, step=zb)
        def _(r):
            pltpu.sync_copy(acc_sh.at[pl.ds(row0 + r, zb)],
                            rows_a.at[pl.ds(0, zb)])
            pltpu.sync_copy(rows_a.at[pl.ds(0, zb)],
                            out_hbm.at[cid, pl.ds(row0 + r, zb)])

    return k(y, src_r, dst_r)


def _tc_inv_prescale(degp, t):
    n, d = t.shape

    def body(degp_ref, t_ref, inv_ref, y_ref):
        ones = jnp.ones((_NW, 1), jnp.float32)
        deg = lax.dot_general(degp_ref[...], ones, (((0,), (0,)), ((), ())),
                              precision=lax.Precision.HIGHEST,
                              preferred_element_type=jnp.float32)
        inv = lax.rsqrt(jnp.maximum(deg, 1.0))
        inv_ref[...] = inv
        y_ref[pl.ds(0, n), :] = t_ref[...] * inv
        y_ref[pl.ds(n, 8), :] = jnp.zeros((8, d), jnp.float32)

    return pl.pallas_call(
        body,
        out_shape=[jax.ShapeDtypeStruct((n, 1), jnp.float32),
                   jax.ShapeDtypeStruct((n + 8, d), jnp.float32)],
    )(degp, t)


def _tc_layer_mid(p, inv, t, w, b):
    n, d = t.shape

    def body(p_ref, inv_ref, t_ref, w_ref, b_ref, h_ref, y_ref):
        inv = inv_ref[...]
        agg = (p_ref[0] + p_ref[1]) * inv
        z = lax.dot_general(agg, w_ref[...], (((1,), (0,)), ((), ())),
                            precision=lax.Precision.HIGHEST,
                            preferred_element_type=jnp.float32)
        h = jnp.maximum(z + b_ref[...] + t_ref[...], 0.0)
        h_ref[...] = h
        y_ref[pl.ds(0, n), :] = h * inv
        y_ref[pl.ds(n, 8), :] = jnp.zeros((8, d), jnp.float32)

    return pl.pallas_call(
        body,
        out_shape=[jax.ShapeDtypeStruct((n, d), jnp.float32),
                   jax.ShapeDtypeStruct((n + 8, d), jnp.float32)],
    )(p, inv, t, w, b.reshape(1, d))


def _tc_layer_out(p, inv, h_prev, w, b):
    n, d = h_prev.shape

    def body(p_ref, inv_ref, h_ref, w_ref, b_ref, o_ref):
        agg = (p_ref[0] + p_ref[1]) * inv_ref[...]
        z = lax.dot_general(agg, w_ref[...], (((1,), (0,)), ((), ())),
                            precision=lax.Precision.HIGHEST,
                            preferred_element_type=jnp.float32)
        o_ref[...] = z + b_ref[...] + h_ref[...]

    return pl.pallas_call(
        body,
        out_shape=jax.ShapeDtypeStruct((n, d), jnp.float32),
    )(p, inv, h_prev, w, b.reshape(1, d))


def kernel(T, edge_index, W1, b1, W2, b2):
    n, d = T.shape
    e = edge_index.shape[1]
    chunk = 128
    bchunk = 16
    epw = e // _NW
    blk_edges = bchunk * chunk
    epw_pad = -(-epw // blk_edges) * blk_edges
    nblk = epw_pad // blk_edges
    pad_n = epw_pad - epw
    w_ids = jnp.arange(_NW, dtype=jnp.int32)
    pad_i = jnp.arange(pad_n, dtype=jnp.int32)
    src_pad = (w_ids[:, None] * 997 + pad_i[None, :]) % n
    dst_pad = jnp.broadcast_to((n + w_ids // _NC)[:, None], (_NW, pad_n))
    src_w = jnp.concatenate(
        [edge_index[0].reshape(_NW, epw), src_pad.astype(jnp.int32)], axis=1)
    dst_w = jnp.concatenate(
        [edge_index[1].reshape(_NW, epw), dst_pad.astype(jnp.int32)], axis=1)
    src_r = src_w.reshape(_NW, nblk, bchunk, chunk)
    dst_r = dst_w.reshape(_NW, nblk, bchunk, chunk)
    dst_deg = edge_index[1].reshape(_NW, e // (_NW * _LANES), _LANES)

    degp = _deg_partials(dst_deg, n).reshape(_NW, n)
    inv, y1 = _tc_inv_prescale(degp, T)
    p1 = _sc_aggregate(y1, src_r, dst_r, n)
    h1, y2 = _tc_layer_mid(p1, inv, T, W1, b1)
    p2 = _sc_aggregate(y2, src_r, dst_r, n)
    return _tc_layer_out(p2, inv, h1, W2, b2)


# speedup vs baseline: 1.1765x; 1.0356x over previous
import dataclasses
import functools

import jax
import jax.numpy as jnp
from jax import lax
from jax.experimental import pallas as pl
from jax.experimental.pallas import tpu as pltpu
from jax.experimental.pallas import tpu_sc as plsc

_NC = 2
_NS = 16
_NW = _NC * _NS
_LANES = 16


def _vector_mesh():
    return plsc.VectorSubcoreMesh(core_axis_name="c", subcore_axis_name="s")


def _sc_compiler_params():
    cp = pltpu.CompilerParams()
    if "needs_layout_passes" in pltpu.CompilerParams.__dataclass_fields__:
        cp = dataclasses.replace(cp, needs_layout_passes=False)
    return cp


def _deg_partials(dst_r, n_nodes):
    _, nchunk, chunk = dst_r.shape

    @functools.partial(
        pl.kernel,
        mesh=_vector_mesh(),
        out_type=jax.ShapeDtypeStruct((_NW, 1, n_nodes), jnp.float32),
        compiler_params=_sc_compiler_params(),
        scratch_types=[
            pltpu.VMEM((nchunk, chunk), jnp.int32),
            pltpu.VMEM((1, n_nodes), jnp.float32),
        ],
    )
    def k(dst_hbm, out_hbm, idx_v, hist_v):
        cid = lax.axis_index("c")
        sid = lax.axis_index("s")
        wid = sid * _NC + cid
        pltpu.sync_copy(dst_hbm.at[wid], idx_v)

        @pl.loop(0, n_nodes, step=_LANES)
        def _(i):
            hist_v[0, pl.ds(i, _LANES)] = jnp.zeros((_LANES,), jnp.float32)

        ones = jnp.ones((_LANES,), jnp.float32)
        zrow = jnp.zeros((_LANES,), jnp.int32)

        @pl.loop(0, nchunk)
        def _(j):
            @pl.loop(0, chunk, step=_LANES)
            def _(kk):
                idx = idx_v[j, pl.ds(kk, _LANES)]
                plsc.addupdate_scatter(hist_v, [zrow, idx], ones)

        pltpu.sync_copy(hist_v, out_hbm.at[wid])

    return k(dst_r)


def _sc_aggregate(y, src_r, dst_r, n_nodes):
    _, nblk, bchunk, chunk = dst_r.shape
    d = y.shape[1]
    zb = 80
    rpt_a = -(-(n_nodes // _NS) // zb) * zb
    last_rows = n_nodes - (_NS - 1) * rpt_a
    n_acc = n_nodes + _NS

    @functools.partial(
        pl.kernel,
        mesh=_vector_mesh(),
        out_type=jax.ShapeDtypeStruct((_NC, n_nodes, d), jnp.float32),
        scratch_types=[
            pltpu.VMEM((bchunk, chunk), jnp.int32),
            pltpu.VMEM((bchunk, chunk), jnp.int32),
            pltpu.VMEM((chunk, d), jnp.float32),
            pltpu.VMEM((chunk, d), jnp.float32),
            pltpu.VMEM_SHARED((n_acc, d), jnp.float32),
            pltpu.SemaphoreType.DMA,
        ],
    )
    def k(y_hbm, src_hbm, dst_hbm, out_hbm, src_i, dst_i,
          rows_a, rows_b, acc_sh, gsem):
        cid = lax.axis_index("c")
        sid = lax.axis_index("s")
        wid = sid * _NC + cid

        row0 = sid * rpt_a
        my_rows = jnp.where(sid == _NS - 1, last_rows, rpt_a)

        @pl.loop(0, zb)
        def _(r):
            @pl.loop(0, d, step=_LANES)
            def _(cc):
                rows_a[r, pl.ds(cc, _LANES)] = jnp.zeros((_LANES,), jnp.float32)

        @pl.loop(0, my_rows, step=zb)
        def _(r):
            pltpu.sync_copy(rows_a.at[pl.ds(0, zb)],
                            acc_sh.at[pl.ds(row0 + r, zb)])

        plsc.subcore_barrier()

        def start_gather(j, buf):
            pltpu.async_copy(y_hbm.at[src_i.at[j]], buf, gsem)

        def wait_gather(buf):
            pltpu.make_async_copy(y_hbm.at[src_i.at[0]], buf, gsem).wait()

        def scatter(j, buf):
            pltpu.sync_copy(buf, acc_sh.at[dst_i.at[j]], add=True)

        @pl.loop(0, nblk)
        def _(b):
            pltpu.sync_copy(src_hbm.at[wid, b], src_i)
            pltpu.sync_copy(dst_hbm.at[wid, b], dst_i)
            start_gather(0, rows_a)

            @pl.loop(0, (bchunk - 2) // 2)
            def _(j2):
                c0 = 2 * j2
                wait_gather(rows_a)
                start_gather(c0 + 1, rows_b)
                scatter(c0, rows_a)
                wait_gather(rows_b)
                start_gather(c0 + 2, rows_a)
                scatter(c0 + 1, rows_b)

            wait_gather(rows_a)
            start_gather(bchunk - 1, rows_b)
            scatter(bchunk - 2, rows_a)
            wait_gather(rows_b)
            scatter(bchunk - 1, rows_b)

        plsc.subcore_barrier()

        @pl.loop(0, my_rows, step=zb)
        def _(r):
            pltpu.sync_copy(acc_sh.at[pl.ds(row0 + r, zb)],
                            out_hbm.at[cid, pl.ds(row0 + r, zb)])

    return k(y, src_r, dst_r)


def _tc_inv_prescale(degp, t):
    n, d = t.shape

    def body(degp_ref, t_ref, inv_ref, y_ref):
        ones = jnp.ones((_NW, 1), jnp.float32)
        deg = lax.dot_general(degp_ref[...], ones, (((0,), (0,)), ((), ())),
                              precision=lax.Precision.HIGHEST,
                              preferred_element_type=jnp.float32)
        inv = lax.rsqrt(jnp.maximum(deg, 1.0))
        inv_ref[...] = inv
        y_ref[pl.ds(0, n), :] = t_ref[...] * inv
        y_ref[pl.ds(n, 8), :] = jnp.zeros((8, d), jnp.float32)

    return pl.pallas_call(
        body,
        out_shape=[jax.ShapeDtypeStruct((n, 1), jnp.float32),
                   jax.ShapeDtypeStruct((n + 8, d), jnp.float32)],
    )(degp, t)


def _tc_layer_mid(p, inv, t, w, b):
    n, d = t.shape

    def body(p_ref, inv_ref, t_ref, w_ref, b_ref, h_ref, y_ref):
        inv = inv_ref[...]
        agg = (p_ref[0] + p_ref[1]) * inv
        z = lax.dot_general(agg, w_ref[...], (((1,), (0,)), ((), ())),
                            precision=lax.Precision.HIGHEST,
                            preferred_element_type=jnp.float32)
        h = jnp.maximum(z + b_ref[...] + t_ref[...], 0.0)
        h_ref[...] = h
        y_ref[pl.ds(0, n), :] = h * inv
        y_ref[pl.ds(n, 8), :] = jnp.zeros((8, d), jnp.float32)

    return pl.pallas_call(
        body,
        out_shape=[jax.ShapeDtypeStruct((n, d), jnp.float32),
                   jax.ShapeDtypeStruct((n + 8, d), jnp.float32)],
    )(p, inv, t, w, b.reshape(1, d))


def _tc_layer_out(p, inv, h_prev, w, b):
    n, d = h_prev.shape

    def body(p_ref, inv_ref, h_ref, w_ref, b_ref, o_ref):
        agg = (p_ref[0] + p_ref[1]) * inv_ref[...]
        z = lax.dot_general(agg, w_ref[...], (((1,), (0,)), ((), ())),
                            precision=lax.Precision.HIGHEST,
                            preferred_element_type=jnp.float32)
        o_ref[...] = z + b_ref[...] + h_ref[...]

    return pl.pallas_call(
        body,
        out_shape=jax.ShapeDtypeStruct((n, d), jnp.float32),
    )(p, inv, h_prev, w, b.reshape(1, d))


def kernel(T, edge_index, W1, b1, W2, b2):
    n, d = T.shape
    e = edge_index.shape[1]
    chunk = 128
    bchunk = 40
    epw = e // _NW
    blk_edges = bchunk * chunk
    epw_pad = -(-epw // blk_edges) * blk_edges
    nblk = epw_pad // blk_edges
    pad_n = epw_pad - epw
    w_ids = jnp.arange(_NW, dtype=jnp.int32)
    pad_i = jnp.arange(pad_n, dtype=jnp.int32)
    src_pad = (w_ids[:, None] * 997 + pad_i[None, :]) % n
    dst_pad = jnp.broadcast_to((n + w_ids // _NC)[:, None], (_NW, pad_n))
    src_w = jnp.concatenate(
        [edge_index[0].reshape(_NW, epw), src_pad.astype(jnp.int32)], axis=1)
    dst_w = jnp.concatenate(
        [edge_index[1].reshape(_NW, epw), dst_pad.astype(jnp.int32)], axis=1)
    src_r = src_w.reshape(_NW, nblk, bchunk, chunk)
    dst_r = dst_w.reshape(_NW, nblk, bchunk, chunk)
    dst_deg = edge_index[1].reshape(_NW, e // (_NW * _LANES), _LANES)

    degp = _deg_partials(dst_deg, n).reshape(_NW, n)
    inv, y1 = _tc_inv_prescale(degp, T)
    p1 = _sc_aggregate(y1, src_r, dst_r, n)
    h1, y2 = _tc_layer_mid(p1, inv, T, W1, b1)
    p2 = _sc_aggregate(y2, src_r, dst_r, n)
    return _tc_layer_out(p2, inv, h1, W2, b2)


# speedup vs baseline: 1.2999x; 1.1049x over previous
import dataclasses
import functools

import jax
import jax.numpy as jnp
from jax import lax
from jax.experimental import pallas as pl
from jax.experimental.pallas import tpu as pltpu
from jax.experimental.pallas import tpu_sc as plsc

_NC = 2
_NS = 16
_NW = _NC * _NS
_LANES = 16


def _vector_mesh():
    return plsc.VectorSubcoreMesh(core_axis_name="c", subcore_axis_name="s")


def _sc_compiler_params():
    cp = pltpu.CompilerParams()
    if "needs_layout_passes" in pltpu.CompilerParams.__dataclass_fields__:
        cp = dataclasses.replace(cp, needs_layout_passes=False)
    return cp


def _deg_partials(dst_r, n_nodes):
    _, nchunk, chunk = dst_r.shape

    @functools.partial(
        pl.kernel,
        mesh=_vector_mesh(),
        out_type=jax.ShapeDtypeStruct((_NW, 1, n_nodes), jnp.float32),
        compiler_params=_sc_compiler_params(),
        scratch_types=[
            pltpu.VMEM((nchunk, chunk), jnp.int32),
            pltpu.VMEM((1, n_nodes), jnp.float32),
        ],
    )
    def k(dst_hbm, out_hbm, idx_v, hist_v):
        cid = lax.axis_index("c")
        sid = lax.axis_index("s")
        wid = sid * _NC + cid
        pltpu.sync_copy(dst_hbm.at[wid], idx_v)

        @pl.loop(0, n_nodes, step=_LANES)
        def _(i):
            hist_v[0, pl.ds(i, _LANES)] = jnp.zeros((_LANES,), jnp.float32)

        ones = jnp.ones((_LANES,), jnp.float32)
        zrow = jnp.zeros((_LANES,), jnp.int32)

        @pl.loop(0, nchunk)
        def _(j):
            @pl.loop(0, chunk, step=_LANES)
            def _(kk):
                idx = idx_v[j, pl.ds(kk, _LANES)]
                plsc.addupdate_scatter(hist_v, [zrow, idx], ones)

        pltpu.sync_copy(hist_v, out_hbm.at[wid])

    return k(dst_r)


def _sc_aggregate(y, src_r, dst_r, n_nodes):
    _, nblk, bchunk, chunk = dst_r.shape
    d = y.shape[1]
    zb = 80
    rpt_a = -(-(n_nodes // _NS) // zb) * zb
    last_rows = n_nodes - (_NS - 1) * rpt_a
    n_acc = n_nodes + _NS

    @functools.partial(
        pl.kernel,
        mesh=_vector_mesh(),
        out_type=jax.ShapeDtypeStruct((_NC, n_nodes, d), jnp.float32),
        scratch_types=[
            pltpu.VMEM((bchunk, chunk), jnp.int32),
            pltpu.VMEM((bchunk, chunk), jnp.int32),
            pltpu.VMEM((chunk, d), jnp.float32),
            pltpu.VMEM((chunk, d), jnp.float32),
            pltpu.VMEM_SHARED((n_acc, d), jnp.float32),
            pltpu.SemaphoreType.DMA,
            pltpu.SemaphoreType.DMA,
        ],
    )
    def k(y_hbm, src_hbm, dst_hbm, out_hbm, src_i, dst_i,
          rows_a, rows_b, acc_sh, gsem, ssem):
        cid = lax.axis_index("c")
        sid = lax.axis_index("s")
        wid = sid * _NC + cid

        row0 = sid * rpt_a
        my_rows = jnp.where(sid == _NS - 1, last_rows, rpt_a)

        @pl.loop(0, zb)
        def _(r):
            @pl.loop(0, d, step=_LANES)
            def _(cc):
                rows_a[r, pl.ds(cc, _LANES)] = jnp.zeros((_LANES,), jnp.float32)

        @pl.loop(0, my_rows, step=zb)
        def _(r):
            pltpu.sync_copy(rows_a.at[pl.ds(0, zb)],
                            acc_sh.at[pl.ds(row0 + r, zb)])

        plsc.subcore_barrier()

        def start_gather(j, buf):
            pltpu.async_copy(y_hbm.at[src_i.at[j]], buf, gsem)

        def wait_gather(buf):
            pltpu.make_async_copy(y_hbm.at[src_i.at[0]], buf, gsem).wait()

        def start_scatter(j, buf):
            pltpu.make_async_copy(buf, acc_sh.at[dst_i.at[j]],
                                  ssem).start(add=True)

        def wait_scatter(buf):
            pltpu.make_async_copy(buf, acc_sh.at[dst_i.at[0]], ssem).wait()

        @pl.loop(0, nblk)
        def _(b):
            pltpu.sync_copy(src_hbm.at[wid, b], src_i)
            pltpu.sync_copy(dst_hbm.at[wid, b], dst_i)
            start_gather(0, rows_a)
            start_gather(1, rows_b)

            @pl.loop(0, (bchunk - 2) // 2)
            def _(j2):
                c0 = 2 * j2
                wait_gather(rows_a)
                start_scatter(c0, rows_a)
                wait_gather(rows_b)
                start_scatter(c0 + 1, rows_b)
                wait_scatter(rows_a)
                start_gather(c0 + 2, rows_a)
                wait_scatter(rows_b)
                start_gather(c0 + 3, rows_b)

            wait_gather(rows_a)
            start_scatter(bchunk - 2, rows_a)
            wait_gather(rows_b)
            start_scatter(bchunk - 1, rows_b)
            wait_scatter(rows_a)
            wait_scatter(rows_b)

        plsc.subcore_barrier()

        @pl.loop(0, my_rows, step=zb)
        def _(r):
            pltpu.sync_copy(acc_sh.at[pl.ds(row0 + r, zb)],
                            out_hbm.at[cid, pl.ds(row0 + r, zb)])

    return k(y, src_r, dst_r)


def _tc_inv_prescale(degp, t):
    n, d = t.shape

    def body(degp_ref, t_ref, inv_ref, y_ref):
        ones = jnp.ones((_NW, 1), jnp.float32)
        deg = lax.dot_general(degp_ref[...], ones, (((0,), (0,)), ((), ())),
                              precision=lax.Precision.HIGHEST,
                              preferred_element_type=jnp.float32)
        inv = lax.rsqrt(jnp.maximum(deg, 1.0))
        inv_ref[...] = inv
        y_ref[pl.ds(0, n), :] = t_ref[...] * inv
        y_ref[pl.ds(n, 8), :] = jnp.zeros((8, d), jnp.float32)

    return pl.pallas_call(
        body,
        out_shape=[jax.ShapeDtypeStruct((n, 1), jnp.float32),
                   jax.ShapeDtypeStruct((n + 8, d), jnp.float32)],
    )(degp, t)


def _tc_layer_mid(p, inv, t, w, b):
    n, d = t.shape

    def body(p_ref, inv_ref, t_ref, w_ref, b_ref, h_ref, y_ref):
        inv = inv_ref[...]
        agg = (p_ref[0] + p_ref[1]) * inv
        z = lax.dot_general(agg, w_ref[...], (((1,), (0,)), ((), ())),
                            precision=lax.Precision.HIGHEST,
                            preferred_element_type=jnp.float32)
        h = jnp.maximum(z + b_ref[...] + t_ref[...], 0.0)
        h_ref[...] = h
        y_ref[pl.ds(0, n), :] = h * inv
        y_ref[pl.ds(n, 8), :] = jnp.zeros((8, d), jnp.float32)

    return pl.pallas_call(
        body,
        out_shape=[jax.ShapeDtypeStruct((n, d), jnp.float32),
                   jax.ShapeDtypeStruct((n + 8, d), jnp.float32)],
    )(p, inv, t, w, b.reshape(1, d))


def _tc_layer_out(p, inv, h_prev, w, b):
    n, d = h_prev.shape

    def body(p_ref, inv_ref, h_ref, w_ref, b_ref, o_ref):
        agg = (p_ref[0] + p_ref[1]) * inv_ref[...]
        z = lax.dot_general(agg, w_ref[...], (((1,), (0,)), ((), ())),
                            precision=lax.Precision.HIGHEST,
                            preferred_element_type=jnp.float32)
        o_ref[...] = z + b_ref[...] + h_ref[...]

    return pl.pallas_call(
        body,
        out_shape=jax.ShapeDtypeStruct((n, d), jnp.float32),
    )(p, inv, h_prev, w, b.reshape(1, d))


def kernel(T, edge_index, W1, b1, W2, b2):
    n, d = T.shape
    e = edge_index.shape[1]
    chunk = 128
    bchunk = 40
    epw = e // _NW
    blk_edges = bchunk * chunk
    epw_pad = -(-epw // blk_edges) * blk_edges
    nblk = epw_pad // blk_edges
    pad_n = epw_pad - epw
    w_ids = jnp.arange(_NW, dtype=jnp.int32)
    pad_i = jnp.arange(pad_n, dtype=jnp.int32)
    src_pad = (w_ids[:, None] * 997 + pad_i[None, :]) % n
    dst_pad = jnp.broadcast_to((n + w_ids // _NC)[:, None], (_NW, pad_n))
    src_w = jnp.concatenate(
        [edge_index[0].reshape(_NW, epw), src_pad.astype(jnp.int32)], axis=1)
    dst_w = jnp.concatenate(
        [edge_index[1].reshape(_NW, epw), dst_pad.astype(jnp.int32)], axis=1)
    src_r = src_w.reshape(_NW, nblk, bchunk, chunk)
    dst_r = dst_w.reshape(_NW, nblk, bchunk, chunk)
    dst_deg = edge_index[1].reshape(_NW, e // (_NW * _LANES), _LANES)

    degp = _deg_partials(dst_deg, n).reshape(_NW, n)
    inv, y1 = _tc_inv_prescale(degp, T)
    p1 = _sc_aggregate(y1, src_r, dst_r, n)
    h1, y2 = _tc_layer_mid(p1, inv, T, W1, b1)
    p2 = _sc_aggregate(y2, src_r, dst_r, n)
    return _tc_layer_out(p2, inv, h1, W2, b2)
